# Initial kernel scaffold; baseline (speedup 1.0000x reference)
import jax
import jax.numpy as jnp
from jax.experimental import pallas as pl


def kernel(context_ids, seq_ids, row_ids, ctx_table, seq_table, Wc1, bc1, Wc2, bc2, Wa0_1, ba0_1, Wa0_2, ba0_2, Wa1_1, ba1_1, Wa1_2, ba1_2):
    raise NotImplementedError("write your pallas kernel here")


# speedup vs baseline: 1.9885x; 1.9885x over previous
import functools

import jax
import jax.numpy as jnp
from jax import lax
from jax.experimental import pallas as pl
from jax.experimental.pallas import tpu as pltpu
from jax.experimental.pallas import tpu_sc as plsc

B = 8
T = 8192
D = 64
NC = 8
NS = 8
H = 512
OUT = 128

SEQ_N = T * NS
CTX_N = B * NC
NUM_WORKERS = 32
ROWS_PER_W = SEQ_N // NUM_WORKERS
CHUNK = 128
N_CHUNKS = ROWS_PER_W // CHUNK

TM = 512
GRID = T // TM


def _sc_gather_body(seq_tab, ctx_tab, seq_ids2d, ctx_ids2d, seq_out, ctx_out,
                    idx_v, rows_a, rows_b, cidx_v, crows_v, sem):
    c = lax.axis_index("c")
    s = lax.axis_index("s")
    wid = s * 2 + c
    base = wid * ROWS_PER_W
    pltpu.sync_copy(seq_ids2d.at[pl.ds(wid * N_CHUNKS, N_CHUNKS)], idx_v)
    bufs = (rows_a, rows_b)
    cps = []
    for k in range(2):
        cps.append(pltpu.async_copy(seq_tab.at[idx_v.at[k]], bufs[k % 2], sem))
    for k in range(N_CHUNKS):
        cps[k].wait()
        pltpu.sync_copy(bufs[k % 2], seq_out.at[pl.ds(base + k * CHUNK, CHUNK)])
        if k + 2 < N_CHUNKS:
            cps.append(pltpu.async_copy(seq_tab.at[idx_v.at[k + 2]],
                                        bufs[k % 2], sem))

    @pl.when(wid == 0)
    def _():
        pltpu.sync_copy(ctx_ids2d, cidx_v)
        pltpu.async_copy(ctx_tab.at[cidx_v.at[0]], crows_v, sem).wait()
        pltpu.sync_copy(crows_v, ctx_out)


@functools.cache
def _sc_gather():
    return pl.kernel(
        _sc_gather_body,
        out_type=[
            jax.ShapeDtypeStruct((SEQ_N, D), jnp.float32),
            jax.ShapeDtypeStruct((CTX_N, D), jnp.float32),
        ],
        mesh=plsc.VectorSubcoreMesh(core_axis_name="c", subcore_axis_name="s"),
        compiler_params=pltpu.CompilerParams(use_tc_tiling_on_sc=False),
        scratch_types=[
            pltpu.VMEM((N_CHUNKS, CHUNK), jnp.int32),
            pltpu.VMEM((CHUNK, D), jnp.float32),
            pltpu.VMEM((CHUNK, D), jnp.float32),
            pltpu.VMEM((1, CTX_N), jnp.int32),
            pltpu.VMEM((CTX_N, D), jnp.float32),
            pltpu.SemaphoreType.DMA,
        ],
    )


def _tc_body(x_ref, ctx_ref, rid_ref, wc1, bc1, wc2, bc2,
             w01, b01, w02, b02, w11, b11, w12, b12,
             ce_ref, ae_ref, sc_ref):
    @pl.when(pl.program_id(0) == 0)
    def _():
        hc = jnp.maximum(
            jnp.dot(ctx_ref[...], wc1[...],
                    preferred_element_type=jnp.float32) + bc1[...], 0.0)
        ce_ref[...] = jnp.dot(hc, wc2[...],
                              preferred_element_type=jnp.float32) + bc2[...]

    x = x_ref[...]
    h0 = jnp.maximum(
        jnp.dot(x, w01[...], preferred_element_type=jnp.float32) + b01[...],
        0.0)
    a0 = jnp.dot(h0, w02[...], preferred_element_type=jnp.float32) + b02[...]
    h1 = jnp.maximum(
        jnp.dot(x, w11[...], preferred_element_type=jnp.float32) + b11[...],
        0.0)
    a1 = jnp.dot(h1, w12[...], preferred_element_type=jnp.float32) + b12[...]
    ae_ref[0] = a0
    ae_ref[1] = a1

    ce = ce_ref[...]
    onehot = (rid_ref[...] == lax.broadcasted_iota(jnp.int32, (TM, B), 1)
              ).astype(jnp.float32)
    aligned = jnp.dot(onehot, ce, preferred_element_type=jnp.float32)
    s0 = jnp.sum(aligned * a0, axis=-1, keepdims=True)
    s1 = jnp.sum(aligned * a1, axis=-1, keepdims=True)
    sc_ref[...] = jnp.concatenate([s0, s1], axis=1)


_tc_grid_spec = dict(
    grid=(GRID,),
    in_specs=[
        pl.BlockSpec((TM, NS * D), lambda i: (i, 0)),
        pl.BlockSpec((B, NC * D), lambda i: (0, 0)),
        pl.BlockSpec((TM, 1), lambda i: (i, 0)),
        pl.BlockSpec((NC * D, H), lambda i: (0, 0)),
        pl.BlockSpec((1, H), lambda i: (0, 0)),
        pl.BlockSpec((H, OUT), lambda i: (0, 0)),
        pl.BlockSpec((1, OUT), lambda i: (0, 0)),
        pl.BlockSpec((NS * D, H), lambda i: (0, 0)),
        pl.BlockSpec((1, H), lambda i: (0, 0)),
        pl.BlockSpec((H, OUT), lambda i: (0, 0)),
        pl.BlockSpec((1, OUT), lambda i: (0, 0)),
        pl.BlockSpec((NS * D, H), lambda i: (0, 0)),
        pl.BlockSpec((1, H), lambda i: (0, 0)),
        pl.BlockSpec((H, OUT), lambda i: (0, 0)),
        pl.BlockSpec((1, OUT), lambda i: (0, 0)),
    ],
    out_specs=[
        pl.BlockSpec((B, OUT), lambda i: (0, 0)),
        pl.BlockSpec((2, TM, OUT), lambda i: (0, i, 0)),
        pl.BlockSpec((TM, 2), lambda i: (i, 0)),
    ],
)


def kernel(context_ids, seq_ids, row_ids, ctx_table, seq_table,
           Wc1, bc1, Wc2, bc2,
           Wa0_1, ba0_1, Wa0_2, ba0_2,
           Wa1_1, ba1_1, Wa1_2, ba1_2):
    seq_ids2d = seq_ids.reshape(SEQ_N // CHUNK, CHUNK).astype(jnp.int32)
    ctx_ids2d = context_ids.reshape(1, CTX_N).astype(jnp.int32)
    seq_rows, ctx_rows = _sc_gather()(seq_table, ctx_table,
                                      seq_ids2d, ctx_ids2d)
    x = seq_rows.reshape(T, NS * D)
    ctx_feats = ctx_rows.reshape(B, NC * D)
    rid2d = row_ids.reshape(T, 1).astype(jnp.int32)

    ce, ae, scores = pl.pallas_call(
        _tc_body,
        out_shape=[
            jax.ShapeDtypeStruct((B, OUT), jnp.float32),
            jax.ShapeDtypeStruct((2, T, OUT), jnp.float32),
            jax.ShapeDtypeStruct((T, 2), jnp.float32),
        ],
        compiler_params=pltpu.CompilerParams(
            dimension_semantics=("arbitrary",)),
        **_tc_grid_spec,
    )(x, ctx_feats, rid2d,
      Wc1, bc1.reshape(1, H), Wc2, bc2.reshape(1, OUT),
      Wa0_1, ba0_1.reshape(1, H), Wa0_2, ba0_2.reshape(1, OUT),
      Wa1_1, ba1_1.reshape(1, H), Wa1_2, ba1_2.reshape(1, OUT))
    return ce, ae, scores


# speedup vs baseline: 2.2146x; 1.1137x over previous
import functools

import jax
import jax.numpy as jnp
from jax import lax
from jax.experimental import pallas as pl
from jax.experimental.pallas import tpu as pltpu
from jax.experimental.pallas import tpu_sc as plsc

B = 8
T = 8192
V = 100000
D = 64
NC = 8
NS = 8
H = 512
OUT = 128

PAIRS = T * NS // 2
NUM_WORKERS = 32
PAIRS_PER_W = PAIRS // NUM_WORKERS
CHUNK = 128
N_CHUNKS = PAIRS_PER_W // CHUNK
CTX_N = B * NC

TM = 512
GRID = T // TM


def _sc_gather_body(seq_tab, ev_ids, od_ids, ctx_pair_tab, cpair_ids,
                    seq_out, ctx_slots,
                    idx_ve, idx_vo, buf_ae, buf_ao, buf_be, buf_bo,
                    cidx, cbuf, sem):
    c = lax.axis_index("c")
    s = lax.axis_index("s")
    wid = s * 2 + c
    base = wid * PAIRS_PER_W
    pltpu.sync_copy(ev_ids.at[pl.ds(wid * N_CHUNKS, N_CHUNKS)], idx_ve)
    pltpu.sync_copy(od_ids.at[pl.ds(wid * N_CHUNKS, N_CHUNKS)], idx_vo)

    bufs = ((buf_ae, buf_ao), (buf_be, buf_bo))

    def start(k, buf):
        be, bo = buf
        a = pltpu.async_copy(seq_tab.at[idx_ve.at[k]], be, sem)
        b = pltpu.async_copy(seq_tab.at[idx_vo.at[k]], bo, sem)
        return a, b

    cps = [start(0, bufs[0]), start(1, bufs[1])]
    for k in range(N_CHUNKS):
        a, b = cps[k]
        a.wait()
        b.wait()
        be, bo = bufs[k % 2]
        rows = seq_out.at[pl.ds(base + k * CHUNK, CHUNK)]
        pltpu.sync_copy(be, rows.at[:, pl.ds(0, D)])
        pltpu.sync_copy(bo, rows.at[:, pl.ds(D, D)])
        if k + 2 < N_CHUNKS:
            cps.append(start(k + 2, bufs[k % 2]))

    @pl.when(wid == 0)
    def _():
        pltpu.sync_copy(cpair_ids, cidx)
        pltpu.async_copy(ctx_pair_tab.at[cidx.at[0]], cbuf, sem).wait()
        pltpu.sync_copy(cbuf, ctx_slots)


@functools.cache
def _sc_gather():
    return pl.kernel(
        _sc_gather_body,
        out_type=[
            jax.ShapeDtypeStruct((PAIRS, 2 * D), jnp.float32),
            jax.ShapeDtypeStruct((CTX_N, 2 * D), jnp.float32),
        ],
        mesh=plsc.VectorSubcoreMesh(core_axis_name="c", subcore_axis_name="s"),
        compiler_params=pltpu.CompilerParams(use_tc_tiling_on_sc=False),
        scratch_types=[
            pltpu.VMEM((N_CHUNKS, CHUNK), jnp.int32),
            pltpu.VMEM((N_CHUNKS, CHUNK), jnp.int32),
            pltpu.VMEM((CHUNK, D), jnp.float32),
            pltpu.VMEM((CHUNK, D), jnp.float32),
            pltpu.VMEM((CHUNK, D), jnp.float32),
            pltpu.VMEM((CHUNK, D), jnp.float32),
            pltpu.VMEM((1, CTX_N), jnp.int32),
            pltpu.VMEM((CTX_N, 2 * D), jnp.float32),
            pltpu.SemaphoreType.DMA,
        ],
    )


def _tc_body(x_ref, slots_ref, par_ref, rid_ref, wc1, bc1, wc2, bc2,
             w01, b01, w02, b02, w11, b11, w12, b12,
             ce_ref, ae_ref, sc_ref):
    @pl.when(pl.program_id(0) == 0)
    def _():
        slots = slots_ref[...]
        valid = jnp.where(par_ref[...] == 1,
                          slots[:, D:2 * D], slots[:, 0:D])
        ctx_x = jnp.concatenate(
            [valid[f * B:(f + 1) * B] for f in range(NC)], axis=1)
        hc = jnp.maximum(
            jnp.dot(ctx_x, wc1[...],
                    preferred_element_type=jnp.float32) + bc1[...], 0.0)
        ce_ref[...] = jnp.dot(hc, wc2[...],
                              preferred_element_type=jnp.float32) + bc2[...]

    x = jnp.concatenate([x_ref[0], x_ref[1], x_ref[2], x_ref[3]], axis=1)
    h0 = jnp.maximum(
        jnp.dot(x, w01[...], preferred_element_type=jnp.float32) + b01[...],
        0.0)
    a0 = jnp.dot(h0, w02[...], preferred_element_type=jnp.float32) + b02[...]
    h1 = jnp.maximum(
        jnp.dot(x, w11[...], preferred_element_type=jnp.float32) + b11[...],
        0.0)
    a1 = jnp.dot(h1, w12[...], preferred_element_type=jnp.float32) + b12[...]
    ae_ref[0] = a0
    ae_ref[1] = a1

    ce = ce_ref[...]
    onehot = (rid_ref[...] == lax.broadcasted_iota(jnp.int32, (TM, B), 1)
              ).astype(jnp.float32)
    aligned = jnp.dot(onehot, ce, preferred_element_type=jnp.float32)
    s0 = jnp.sum(aligned * a0, axis=-1, keepdims=True)
    s1 = jnp.sum(aligned * a1, axis=-1, keepdims=True)
    sc_ref[...] = jnp.concatenate([s0, s1], axis=1)


_tc_grid_spec = dict(
    grid=(GRID,),
    in_specs=[
        pl.BlockSpec((4, TM, 128), lambda i: (0, i, 0)),
        pl.BlockSpec((CTX_N, 2 * D), lambda i: (0, 0)),
        pl.BlockSpec((CTX_N, 1), lambda i: (0, 0)),
        pl.BlockSpec((TM, 1), lambda i: (i, 0)),
        pl.BlockSpec((NC * D, H), lambda i: (0, 0)),
        pl.BlockSpec((1, H), lambda i: (0, 0)),
        pl.BlockSpec((H, OUT), lambda i: (0, 0)),
        pl.BlockSpec((1, OUT), lambda i: (0, 0)),
        pl.BlockSpec((NS * D, H), lambda i: (0, 0)),
        pl.BlockSpec((1, H), lambda i: (0, 0)),
        pl.BlockSpec((H, OUT), lambda i: (0, 0)),
        pl.BlockSpec((1, OUT), lambda i: (0, 0)),
        pl.BlockSpec((NS * D, H), lambda i: (0, 0)),
        pl.BlockSpec((1, H), lambda i: (0, 0)),
        pl.BlockSpec((H, OUT), lambda i: (0, 0)),
        pl.BlockSpec((1, OUT), lambda i: (0, 0)),
    ],
    out_specs=[
        pl.BlockSpec((B, OUT), lambda i: (0, 0)),
        pl.BlockSpec((2, TM, OUT), lambda i: (0, i, 0)),
        pl.BlockSpec((TM, 2), lambda i: (i, 0)),
    ],
)


def kernel(context_ids, seq_ids, row_ids, ctx_table, seq_table,
           Wc1, bc1, Wc2, bc2,
           Wa0_1, ba0_1, Wa0_2, ba0_2,
           Wa1_1, ba1_1, Wa1_2, ba1_2):
    sids = seq_ids.astype(jnp.int32)
    ev_ids = sids[:, 0::2].T.reshape(PAIRS // CHUNK, CHUNK)
    od_ids = sids[:, 1::2].T.reshape(PAIRS // CHUNK, CHUNK)

    cflat = context_ids.astype(jnp.int32).T.reshape(-1)
    cpair = (cflat // 2).reshape(1, CTX_N)
    cpar = (cflat % 2).reshape(CTX_N, 1)
    ctx_pair_tab = ctx_table.reshape(V // 2, 2 * D)

    seq_rows, ctx_slots = _sc_gather()(seq_table, ev_ids, od_ids,
                                       ctx_pair_tab, cpair)
    x3d = seq_rows.reshape(4, T, 128)
    rid2d = row_ids.reshape(T, 1).astype(jnp.int32)

    ce, ae, scores = pl.pallas_call(
        _tc_body,
        out_shape=[
            jax.ShapeDtypeStruct((B, OUT), jnp.float32),
            jax.ShapeDtypeStruct((2, T, OUT), jnp.float32),
            jax.ShapeDtypeStruct((T, 2), jnp.float32),
        ],
        compiler_params=pltpu.CompilerParams(
            dimension_semantics=("arbitrary",)),
        **_tc_grid_spec,
    )(x3d, ctx_slots, cpar, rid2d,
      Wc1, bc1.reshape(1, H), Wc2, bc2.reshape(1, OUT),
      Wa0_1, ba0_1.reshape(1, H), Wa0_2, ba0_2.reshape(1, OUT),
      Wa1_1, ba1_1.reshape(1, H), Wa1_2, ba1_2.reshape(1, OUT))
    return ce, ae, scores
